# baseline (speedup 1.0000x reference)
import functools

import jax
import jax.numpy as jnp
from jax.experimental import pallas as pl
from jax.experimental.pallas import tpu as pltpu

N_SITES = 784
D = 128
B = 256
S = 56
NBLK = N_SITES // S


def _deint_perm():
    row = jax.lax.broadcasted_iota(jnp.int32, (2 * D, 2 * D), 0)
    col = jax.lax.broadcasted_iota(jnp.int32, (2 * D, 2 * D), 1)
    return jnp.where((row % 2) * D + row // 2 == col, 1.0, 0.0)


def _mps_body(mint_ref, xft_ref, out_ref, alt_ref, gl_ref, ms_ref):
    j = pl.program_id(0)

    @pl.when(j == 0)
    def _init():
        row = jax.lax.broadcasted_iota(jnp.int32, (D, B), 0)
        alt_ref[...] = jnp.where(row == 0, 1.0, 0.0)
        rowg = jax.lax.broadcasted_iota(jnp.int32, (D, D), 0)
        colg = jax.lax.broadcasted_iota(jnp.int32, (D, D), 1)
        gl_ref[...] = jnp.where((rowg == 0) & (colg == 0), 1.0, 0.0)

    perm = _deint_perm()

    for s in range(S):
        ms_ref[s] = jnp.dot(mint_ref[s], perm,
                            preferred_element_type=jnp.float32)

    xblk = xft_ref[...]
    e0b = jnp.cos(0.5 * jnp.pi * xblk)
    e1b = jnp.sin(0.5 * jnp.pi * xblk)

    alt = alt_ref[...]
    gl = gl_ref[...]
    for s in range(S):
        m = ms_ref[s]
        yv = jax.lax.dot_general(
            m, alt, (((0,), (0,)), ((), ())),
            preferred_element_type=jnp.float32)
        zv = jax.lax.dot_general(
            m, gl, (((0,), (0,)), ((), ())),
            preferred_element_type=jnp.float32)
        alt = yv[:D] * e0b[s:s + 1] + yv[D:] * e1b[s:s + 1]
        r0 = jnp.dot(zv[:D], m[:, :D], preferred_element_type=jnp.float32)
        r1 = jnp.dot(zv[D:], m[:, D:], preferred_element_type=jnp.float32)
        gl = r0 + r1
    alt_ref[...] = alt
    gl_ref[...] = gl

    @pl.when(j == NBLK - 1)
    def _():
        out_ref[0] = alt
        out_ref[1, :, :D] = gl


@functools.partial(jax.jit, static_argnames=("interpret",))
def kernel(x, MPS, interpret=False):
    xft = x.reshape(B, -1).T
    mint = MPS.reshape(N_SITES, D, 2 * D)

    buf = pl.pallas_call(
        _mps_body,
        grid=(NBLK,),
        in_specs=[
            pl.BlockSpec((S, D, 2 * D), lambda j: (j, 0, 0)),
            pl.BlockSpec((S, B), lambda j: (j, 0)),
        ],
        out_specs=pl.BlockSpec((2, D, B), lambda j: (0, 0, 0)),
        out_shape=jax.ShapeDtypeStruct((2, D, B), jnp.float32),
        scratch_shapes=[
            pltpu.VMEM((D, B), jnp.float32),
            pltpu.VMEM((D, D), jnp.float32),
            pltpu.VMEM((S, D, 2 * D), jnp.float32),
        ],
        compiler_params=pltpu.CompilerParams(
            dimension_semantics=("arbitrary",),
        ),
        interpret=interpret,
    )(mint, xft)

    amp = buf[0, 0, :]
    norm_sq = buf[1, 0, 0]
    return amp * amp / norm_sq


# speedup vs baseline: 1.1274x; 1.1274x over previous
import functools

import jax
import jax.numpy as jnp
from jax.experimental import pallas as pl
from jax.experimental.pallas import tpu as pltpu

N_SITES = 784
D = 128
B = 256
S = 56
NBLK = N_SITES // S


def _mps_body(mcat_ref, xft_ref, out_ref, alt_ref, gl_ref):
    j = pl.program_id(0)

    @pl.when(j == 0)
    def _init():
        row = jax.lax.broadcasted_iota(jnp.int32, (D, B), 0)
        alt_ref[...] = jnp.where(row == 0, 1.0, 0.0)
        rowg = jax.lax.broadcasted_iota(jnp.int32, (D, D), 0)
        colg = jax.lax.broadcasted_iota(jnp.int32, (D, D), 1)
        gl_ref[...] = jnp.where((rowg == 0) & (colg == 0), 1.0, 0.0)

    xblk = xft_ref[...]
    e0b = jnp.cos(0.5 * jnp.pi * xblk)
    e1b = jnp.sin(0.5 * jnp.pi * xblk)

    alt = alt_ref[...]
    gl = gl_ref[...]
    for s in range(S):
        m = mcat_ref[s]
        yv = jax.lax.dot_general(
            m, alt, (((0,), (0,)), ((), ())),
            preferred_element_type=jnp.float32)
        zv = jax.lax.dot_general(
            m, gl, (((0,), (0,)), ((), ())),
            preferred_element_type=jnp.float32)
        alt = yv[:D] * e0b[s:s + 1] + yv[D:] * e1b[s:s + 1]
        r0 = jnp.dot(zv[:D], m[:, :D], preferred_element_type=jnp.float32)
        r1 = jnp.dot(zv[D:], m[:, D:], preferred_element_type=jnp.float32)
        gl = r0 + r1
    alt_ref[...] = alt
    gl_ref[...] = gl

    @pl.when(j == NBLK - 1)
    def _():
        out_ref[0] = alt
        out_ref[1, :, :D] = gl


@functools.partial(jax.jit, static_argnames=("interpret",))
def kernel(x, MPS, interpret=False):
    xft = x.reshape(B, -1).T
    mcat = MPS.transpose(0, 1, 3, 2).reshape(N_SITES, D, 2 * D)

    buf = pl.pallas_call(
        _mps_body,
        grid=(NBLK,),
        in_specs=[
            pl.BlockSpec((S, D, 2 * D), lambda j: (j, 0, 0)),
            pl.BlockSpec((S, B), lambda j: (j, 0)),
        ],
        out_specs=pl.BlockSpec((2, D, B), lambda j: (0, 0, 0)),
        out_shape=jax.ShapeDtypeStruct((2, D, B), jnp.float32),
        scratch_shapes=[
            pltpu.VMEM((D, B), jnp.float32),
            pltpu.VMEM((D, D), jnp.float32),
        ],
        compiler_params=pltpu.CompilerParams(
            dimension_semantics=("arbitrary",),
        ),
        interpret=interpret,
    )(mcat, xft)

    amp = buf[0, 0, :]
    norm_sq = buf[1, 0, 0]
    return amp * amp / norm_sq


# speedup vs baseline: 1.7137x; 1.5201x over previous
import functools

import jax
import jax.numpy as jnp
from jax.experimental import pallas as pl
from jax.experimental.pallas import tpu as pltpu

N_SITES = 784
D = 128
B = 256
S = 56
NBLK = N_SITES // S


def _row_sort_perm():
    row = jax.lax.broadcasted_iota(jnp.int32, (2 * D, 2 * D), 0)
    col = jax.lax.broadcasted_iota(jnp.int32, (2 * D, 2 * D), 1)
    return jnp.where(2 * (row % D) + row // D == col, 1.0, 0.0)


def _mps_body(m2_ref, xft_ref, out_ref, alt_ref, gl_ref):
    j = pl.program_id(0)

    @pl.when(j == 0)
    def _init():
        row = jax.lax.broadcasted_iota(jnp.int32, (D, B), 0)
        alt_ref[...] = jnp.where(row == 0, 1.0, 0.0)
        rowg = jax.lax.broadcasted_iota(jnp.int32, (D, D), 0)
        colg = jax.lax.broadcasted_iota(jnp.int32, (D, D), 1)
        gl_ref[...] = jnp.where((rowg == 0) & (colg == 0), 1.0, 0.0)

    perm = _row_sort_perm()

    xblk = xft_ref[...]
    e0b = jnp.cos(0.5 * jnp.pi * xblk)
    e1b = jnp.sin(0.5 * jnp.pi * xblk)

    alt = alt_ref[...]
    gl = gl_ref[...]
    for s in range(S):
        slab = m2_ref[2 * D * s:2 * D * (s + 1), :]
        mv = jnp.dot(perm, slab,
                     preferred_element_type=jnp.float32)
        mcat = jnp.concatenate([mv[:D], mv[D:]], axis=1)

        vb = jnp.concatenate([alt * e0b[s:s + 1], alt * e1b[s:s + 1]], axis=0)
        alt = jax.lax.dot_general(
            mv, vb, (((0,), (0,)), ((), ())),
            preferred_element_type=jnp.float32)

        w = jnp.dot(gl, mcat,
                    preferred_element_type=jnp.float32)
        wv = jnp.concatenate([w[:, :D], w[:, D:]], axis=0)
        gl = jax.lax.dot_general(
            wv, mv, (((0,), (0,)), ((), ())),
            preferred_element_type=jnp.float32)
    alt_ref[...] = alt
    gl_ref[...] = gl

    @pl.when(j == NBLK - 1)
    def _():
        out_ref[0] = alt
        out_ref[1, :, :D] = gl


@functools.partial(jax.jit, static_argnames=("interpret",))
def kernel(x, MPS, interpret=False):
    xft = x.reshape(B, -1).T
    m2 = MPS.transpose(0, 1, 3, 2).reshape(N_SITES * 2 * D, D)

    buf = pl.pallas_call(
        _mps_body,
        grid=(NBLK,),
        in_specs=[
            pl.BlockSpec((S * 2 * D, D), lambda j: (j, 0)),
            pl.BlockSpec((S, B), lambda j: (j, 0)),
        ],
        out_specs=pl.BlockSpec((2, D, B), lambda j: (0, 0, 0)),
        out_shape=jax.ShapeDtypeStruct((2, D, B), jnp.float32),
        scratch_shapes=[
            pltpu.VMEM((D, B), jnp.float32),
            pltpu.VMEM((D, D), jnp.float32),
        ],
        compiler_params=pltpu.CompilerParams(
            dimension_semantics=("arbitrary",),
        ),
        interpret=interpret,
    )(m2, xft)

    amp = buf[0, 0, :]
    norm_sq = buf[1, 0, 0]
    return amp * amp / norm_sq
